# baseline (speedup 1.0000x reference)
import jax
import jax.numpy as jnp
from jax import lax
from jax.experimental import pallas as pl
from jax.experimental.pallas import tpu as pltpu
from jax.experimental.pallas import tpu_sc as plsc

VOCAB = 100000
D = 1024
B = 4 * 4096
NC = 2
NS = 16
NW = NC * NS
PER_W = B // NW
C = 16
NCHUNK = PER_W // C
NBUF = 3
LANES = 16
SCALE = 32.0


def _sc_body(idx_hbm, table_hbm, out_hbm, idx_v, in_buf, out_buf, *sems):
    gsems = sems[:NBUF]
    osems = sems[NBUF:]
    wid = lax.axis_index("s") * NC + lax.axis_index("c")
    row0 = wid * PER_W

    pltpu.sync_copy(idx_hbm.at[wid], idx_v)

    def start_gather(g, s):
        pltpu.async_copy(table_hbm.at[idx_v.at[g]], in_buf.at[s], gsems[s])

    def wait_gather(g, s):
        pltpu.make_async_copy(
            table_hbm.at[idx_v.at[g]], in_buf.at[s], gsems[s]).wait()

    def start_out(g, s):
        pltpu.async_copy(
            out_buf.at[s], out_hbm.at[pl.ds(row0 + g * C, C)], osems[s])

    def wait_out(g, s):
        pltpu.make_async_copy(
            out_buf.at[s], out_hbm.at[pl.ds(row0 + g * C, C)], osems[s]).wait()

    def scale(s):
        @pl.loop(0, C)
        def _(r):
            for c in range(D // LANES):
                sl = pl.ds(c * LANES, LANES)
                out_buf[s, r, sl] = in_buf[s, r, sl] * SCALE

    for s in range(NBUF):
        start_gather(s, s)

    for g in range(NBUF):
        s = g % NBUF
        wait_gather(g, s)
        scale(s)
        start_gather(g + NBUF, s)
        start_out(g, s)

    n_groups = NCHUNK // NBUF - 2
    @pl.loop(1, 1 + n_groups)
    def _(p):
        for s in range(NBUF):
            g = p * NBUF + s
            wait_gather(g, s)
            wait_out(g - NBUF, s)
            scale(s)
            start_gather(g + NBUF, s)
            start_out(g, s)

    for g in range((1 + n_groups) * NBUF, NCHUNK):
        s = g % NBUF
        wait_gather(g, s)
        wait_out(g - NBUF, s)
        scale(s)
        if g + NBUF < NCHUNK:
            start_gather(g + NBUF, s)
        start_out(g, s)

    for g in range(NCHUNK - NBUF, NCHUNK):
        wait_out(g, g % NBUF)


def kernel(x, table):
    idx = x.reshape(NW, NCHUNK, C).astype(jnp.int32)
    call = pl.kernel(
        _sc_body,
        out_type=jax.ShapeDtypeStruct((B, D), jnp.float32),
        mesh=plsc.VectorSubcoreMesh(
            core_axis_name="c", subcore_axis_name="s",
            num_cores=NC, num_subcores=NS),
        scratch_types=[
            pltpu.VMEM((NCHUNK, C), jnp.int32),
            pltpu.VMEM((NBUF, C, D), jnp.float32),
            pltpu.VMEM((NBUF, C, D), jnp.float32),
        ] + [pltpu.SemaphoreType.DMA] * (2 * NBUF),
    )
    out = call(idx, table)
    return out.reshape(x.shape + (D,))


# speedup vs baseline: 1.3904x; 1.3904x over previous
import jax
import jax.numpy as jnp
from jax import lax
from jax.experimental import pallas as pl
from jax.experimental.pallas import tpu as pltpu
from jax.experimental.pallas import tpu_sc as plsc

VOCAB = 100000
D = 1024
B = 4 * 4096
NC = 2
NS = 16
NW = NC * NS
PER_W = B // NW
C = 32
NCHUNK = PER_W // C
NB = 3
LANES = 16
SCALE = 32.0
XCOLS = 4096
W_PER_XROW = XCOLS // PER_W


def _sc_body(idx_hbm, table_hbm, out_hbm, idx_v, buf, *sems):
    gsems = sems[:NB]
    osems = sems[NB:]
    wid = lax.axis_index("s") * NC + lax.axis_index("c")
    row0 = wid * PER_W

    pltpu.sync_copy(
        idx_hbm.at[wid // W_PER_XROW,
                   pl.ds((wid % W_PER_XROW) * PER_W, PER_W)], idx_v)

    def start_gather(g, s):
        pltpu.async_copy(table_hbm.at[idx_v.at[pl.ds(g * C, C)]],
                         buf.at[s], gsems[s])

    def wait_gather(g, s):
        pltpu.make_async_copy(table_hbm.at[idx_v.at[pl.ds(g * C, C)]],
                              buf.at[s], gsems[s]).wait()

    def start_out(g, s):
        pltpu.async_copy(buf.at[s], out_hbm.at[pl.ds(row0 + g * C, C)],
                         osems[s])

    def wait_out(g, s):
        pltpu.make_async_copy(buf.at[s],
                              out_hbm.at[pl.ds(row0 + g * C, C)],
                              osems[s]).wait()

    def scale(s):
        @pl.loop(0, C)
        def _(r):
            for c in range(D // LANES):
                sl = pl.ds(c * LANES, LANES)
                buf[s, r, sl] = buf[s, r, sl] * SCALE

    start_gather(0, 0)
    start_gather(1, 1)
    start_gather(2, 2)

    for g in range(NCHUNK):
        s = g % NB
        wait_gather(g, s)
        scale(s)
        start_out(g, s)
        if g >= 1 and g + 2 < NCHUNK:
            wait_out(g - 1, (g - 1) % NB)
            start_gather(g + 2, (g + 2) % NB)

    for g in range(NCHUNK - 3, NCHUNK):
        wait_out(g, g % NB)


def kernel(x, table):
    call = pl.kernel(
        _sc_body,
        out_type=jax.ShapeDtypeStruct((B, D), jnp.float32),
        mesh=plsc.VectorSubcoreMesh(
            core_axis_name="c", subcore_axis_name="s",
            num_cores=NC, num_subcores=NS),
        scratch_types=[
            pltpu.VMEM((PER_W,), jnp.int32),
            pltpu.VMEM((NB, C, D), jnp.float32),
        ] + [pltpu.SemaphoreType.DMA] * (2 * NB),
    )
    out = call(x.astype(jnp.int32), table)
    return out.reshape(x.shape + (D,))
